# baseline (speedup 1.0000x reference)
import dataclasses
import functools

import numpy as _np

import jax
import jax.numpy as jnp
from jax import lax
from jax.experimental import pallas as pl
from jax.experimental.pallas import tpu as pltpu
from jax.experimental.pallas import tpu_sc as plsc

_N = 10000
_E = 320000
_NG = 16
_OUT = 1200
_W = 128
_NWORK = 32
_WPW = 80
_NWIN = _NWORK * _WPW
_EP = _NWIN * _W
_RPS = 624

_f32 = jnp.float32
_i32 = jnp.int32


def _pre_body(ew_ref, x_ref, w_ref, m_ref, hw_ref):
    m_ref[...] = jnp.broadcast_to(jnp.max(ew_ref[...]), (8, 128))
    hw_ref[...] = jnp.dot(x_ref[...], w_ref[...], preferred_element_type=_f32)


def _scale_split_body(hw_ref, deg0c_ref, deg1c_ref, m_ref, dcol_ref, out_ref):
    inv_m = 1.0 / m_ref[0, 0]
    deg = 1.0 + (deg0c_ref[...] + deg1c_ref[...]) * inv_m
    dcol = lax.rsqrt(deg)
    dcol_ref[...] = dcol
    hws = hw_ref[...] * dcol
    out_ref[0] = hws[:, :64]
    out_ref[1] = hws[:, 64:]


def _layer1_body(acc_ref, hws_ref, dcol_ref, m_ref, b_ref, wn_ref, out_ref):
    inv_m = 1.0 / m_ref[0, 0]
    dcol = dcol_ref[...]
    b = b_ref[...]
    agga = dcol * (acc_ref[0] * inv_m + hws_ref[0]) + b[:64]
    aggb = dcol * (acc_ref[1] * inv_m + hws_ref[1]) + b[64:]
    h = jnp.maximum(jnp.concatenate([agga, aggb], axis=1), 0.0)
    out_ref[...] = jnp.dot(h, wn_ref[...], preferred_element_type=_f32) * dcol


def _layer_body(acc0_ref, acc1_ref, hws_ref, dcol_ref, m_ref, b_ref, wn_ref,
                out_ref):
    inv_m = 1.0 / m_ref[0, 0]
    dcol = dcol_ref[...]
    agg = (dcol * ((acc0_ref[...] + acc1_ref[...]) * inv_m + hws_ref[...])
           + b_ref[...])
    h = jnp.maximum(agg, 0.0)
    out_ref[...] = jnp.dot(h, wn_ref[...], preferred_element_type=_f32) * dcol


def _final_body(acc0_ref, acc1_ref, hws_ref, dcol_ref, m_ref, b_ref, batch_ref,
                wlin_ref, blin_ref, out_ref):
    inv_m = 1.0 / m_ref[0, 0]
    agg = (dcol_ref[...] * ((acc0_ref[...] + acc1_ref[...]) * inv_m
                            + hws_ref[...]) + b_ref[...])
    h = jnp.maximum(agg, 0.0)
    gids = lax.broadcasted_iota(_i32, (_NG, _N), 0)
    oh = (batch_ref[...] == gids).astype(_f32)
    cnt = jnp.sum(oh, axis=1, keepdims=True)
    sums = jnp.dot(oh, h, preferred_element_type=_f32)
    pooled = sums / jnp.maximum(cnt, 1.0)
    out_ref[...] = (jnp.dot(pooled, wlin_ref[...], preferred_element_type=_f32)
                    + blin_ref[...])


def _tc(body, out_shape):
    return pl.pallas_call(body, out_shape=jax.ShapeDtypeStruct(out_shape, _f32))


_MESH = plsc.VectorSubcoreMesh(core_axis_name="c", subcore_axis_name="s")

_GDN = lax.GatherDimensionNumbers(
    offset_dims=(), collapsed_slice_dims=(0,), start_index_map=(0,),
    operand_batching_dims=(), start_indices_batching_dims=())

_CP = pltpu.CompilerParams()
if "needs_layout_passes" in pltpu.CompilerParams.__dataclass_fields__:
    _CP = dataclasses.replace(_CP, needs_layout_passes=False)
if "use_tc_tiling_on_sc" in pltpu.CompilerParams.__dataclass_fields__:
    _CP = dataclasses.replace(_CP, use_tc_tiling_on_sc=False)


def _deg_kernel(dst_hbm, ewn_hbm, out0_hbm, out1_hbm, didx_v, ew_v, zb_v,
                acc_sh, sem):
    cid = lax.axis_index("c")
    sid = lax.axis_index("s")
    wid = sid * 2 + cid

    z16 = jnp.zeros((16,), _f32)

    @pl.loop(0, 125)
    def _(r):
        zb_v[pl.ds(r * 16, 16)] = z16

    @pl.when(sid < 5)
    def _():
        pltpu.sync_copy(zb_v, acc_sh.at[pl.ds(sid * 2000, 2000)])

    pltpu.sync_copy(dst_hbm.at[pl.ds(wid * _WPW, _WPW)], didx_v)
    pltpu.sync_copy(ewn_hbm.at[pl.ds(wid * _WPW, _WPW)], ew_v)
    plsc.subcore_barrier()

    @pl.loop(0, _WPW, step=8)
    def _(w):
        for j in range(8):
            pltpu.async_copy(ew_v.at[w + j], acc_sh.at[didx_v.at[w + j]],
                             sem, add=True)
        for j in range(8):
            pltpu.make_async_copy(ewn_hbm.at[0], ew_v.at[0], sem).wait()

    plsc.subcore_barrier()

    @pl.when(sid < 5)
    def _():
        sl = pl.ds(sid * 2000, 2000)
        pltpu.sync_copy(acc_sh.at[sl], zb_v)

        @pl.when(cid == 0)
        def _():
            pltpu.sync_copy(zb_v, out0_hbm.at[sl])

        @pl.when(cid == 1)
        def _():
            pltpu.sync_copy(zb_v, out1_hbm.at[sl])


def _deg_call(dst, ewn):
    k = pl.kernel(
        _deg_kernel,
        out_type=(jax.ShapeDtypeStruct((_N,), _f32),
                  jax.ShapeDtypeStruct((_N,), _f32)),
        mesh=_MESH,
        compiler_params=_CP,
        scratch_types=[
            pltpu.VMEM((_WPW, _W), _i32),
            pltpu.VMEM((_WPW, _W), _f32),
            pltpu.VMEM((2000,), _f32),
            pltpu.VMEM_SHARED((_N,), _f32),
            pltpu.SemaphoreType.DMA,
        ],
    )
    return k(dst.reshape(_NWIN, _W), ewn.reshape(_NWIN, _W))


def _edge_pipeline(tbl_hbm, src_hbm, dst_hbm, ewn_hbm, woff, nc,
                   sidx_v, didx_v, ew_v, rows, acc_sh, gsem, ssem):
    pltpu.sync_copy(src_hbm.at[pl.ds(woff, _WPW)], sidx_v)
    pltpu.sync_copy(dst_hbm.at[pl.ds(woff, _WPW)], didx_v)
    pltpu.sync_copy(ewn_hbm.at[pl.ds(woff, _WPW)], ew_v)

    def start_gather(w, j):
        pltpu.async_copy(tbl_hbm.at[sidx_v.at[w]], rows[j], gsem[j])

    def start_scatter(w, j):
        pltpu.async_copy(rows[j], acc_sh.at[didx_v.at[w]], ssem[j], add=True)

    def wait(ref, sem):
        pltpu.make_async_copy(tbl_hbm.at[pl.ds(0, _W)], ref, sem).wait()

    def scale(w, j):
        @plsc.parallel_loop(0, _W // 16)
        def _(gi):
            ewv = ew_v[w, pl.ds(gi * 16, 16)]
            for k in range(16):
                g = lax.gather(
                    ewv, jnp.full((16, 1), k, _i32),
                    dimension_numbers=_GDN, slice_sizes=(1,),
                    mode=lax.GatherScatterMode.PROMISE_IN_BOUNDS)
                r = gi * 16 + k
                for c in range(nc):
                    rows[j][r, pl.ds(c * 16, 16)] = (
                        rows[j][r, pl.ds(c * 16, 16)] * g)

    for j in range(4):
        start_gather(j, j)

    @pl.loop(0, _WPW // 4 - 1)
    def _(p):
        w0 = p * 4
        wait(rows[0], gsem[0]); scale(w0 + 0, 0); start_scatter(w0 + 0, 0)
        wait(rows[1], gsem[1]); scale(w0 + 1, 1); start_scatter(w0 + 1, 1)
        wait(rows[0], ssem[0]); start_gather(w0 + 4, 0)
        wait(rows[2], gsem[2]); scale(w0 + 2, 2); start_scatter(w0 + 2, 2)
        wait(rows[1], ssem[1]); start_gather(w0 + 5, 1)
        wait(rows[3], gsem[3]); scale(w0 + 3, 3); start_scatter(w0 + 3, 3)
        wait(rows[2], ssem[2]); start_gather(w0 + 6, 2)
        wait(rows[3], ssem[3]); start_gather(w0 + 7, 3)

    wE = _WPW - 4
    wait(rows[0], gsem[0]); scale(wE + 0, 0); start_scatter(wE + 0, 0)
    wait(rows[1], gsem[1]); scale(wE + 1, 1); start_scatter(wE + 1, 1)
    wait(rows[0], ssem[0])
    wait(rows[2], gsem[2]); scale(wE + 2, 2); start_scatter(wE + 2, 2)
    wait(rows[1], ssem[1])
    wait(rows[3], gsem[3]); scale(wE + 3, 3); start_scatter(wE + 3, 3)
    wait(rows[2], ssem[2])
    wait(rows[3], ssem[3])


def _edge_kernel(F, hws_hbm, src_hbm, dst_hbm, ewn_hbm, out0_hbm, out1_hbm,
                 sidx_v, didx_v, ew_v, r0, r1, r2, r3, zb_v, acc_sh,
                 g0, g1, g2, g3, s0, s1, s2, s3):
    cid = lax.axis_index("c")
    sid = lax.axis_index("s")
    wid = sid * 2 + cid
    nc = F // 16
    rows = (r0, r1, r2, r3)
    gsem = (g0, g1, g2, g3)
    ssem = (s0, s1, s2, s3)

    z16 = jnp.zeros((16,), _f32)

    @pl.loop(0, 104)
    def _(r):
        for c in range(nc):
            zb_v[r, pl.ds(c * 16, 16)] = z16

    @pl.loop(0, 6)
    def _(j):
        pltpu.sync_copy(zb_v, acc_sh.at[pl.ds(sid * _RPS + j * 104, 104)])

    @pl.when(sid == 0)
    def _():
        pltpu.sync_copy(zb_v.at[pl.ds(0, 16)], acc_sh.at[pl.ds(16 * _RPS, 16)])

    _edge_pipeline(hws_hbm, src_hbm, dst_hbm, ewn_hbm, wid * _WPW, nc,
                   sidx_v, didx_v, ew_v, rows, acc_sh, gsem, ssem)
    plsc.subcore_barrier()

    def _dump(out_hbm):
        sl = pl.ds(sid * _RPS, _RPS)
        pltpu.async_copy(acc_sh.at[sl], out_hbm.at[sl], g0).wait()

        @pl.when(sid == 0)
        def _():
            tl = pl.ds(16 * _RPS, 16)
            pltpu.async_copy(acc_sh.at[tl], out_hbm.at[tl], g0).wait()

    @pl.when(cid == 0)
    def _():
        _dump(out0_hbm)

    @pl.when(cid == 1)
    def _():
        _dump(out1_hbm)


def _edge_call(F, hws, src, dst, ewn):
    k = pl.kernel(
        functools.partial(_edge_kernel, F),
        out_type=(jax.ShapeDtypeStruct((_N, F), _f32),
                  jax.ShapeDtypeStruct((_N, F), _f32)),
        mesh=_MESH,
        compiler_params=_CP,
        scratch_types=[
            pltpu.VMEM((_WPW, _W), _i32),
            pltpu.VMEM((_WPW, _W), _i32),
            pltpu.VMEM((_WPW, _W), _f32),
            pltpu.VMEM((_W, F), _f32),
            pltpu.VMEM((_W, F), _f32),
            pltpu.VMEM((_W, F), _f32),
            pltpu.VMEM((_W, F), _f32),
            pltpu.VMEM((104, F), _f32),
            pltpu.VMEM_SHARED((_N, F), _f32),
        ] + [pltpu.SemaphoreType.DMA] * 8,
    )
    return k(hws, src.reshape(_NWIN, _W), dst.reshape(_NWIN, _W),
             ewn.reshape(_NWIN, _W))


def _edge1_kernel(hws_hbm, src_hbm, dst_hbm, ewn_hbm, out_hbm,
                  sidx_v, didx_v, ew_v, r0, r1, r2, r3, zb_v, acc_sh,
                  g0, g1, g2, g3, s0, s1, s2, s3):
    cid = lax.axis_index("c")
    sid = lax.axis_index("s")
    rows = (r0, r1, r2, r3)
    gsem = (g0, g1, g2, g3)
    ssem = (s0, s1, s2, s3)
    z16 = jnp.zeros((16,), _f32)

    @pl.loop(0, 104)
    def _(r):
        for c in range(4):
            zb_v[r, pl.ds(c * 16, 16)] = z16

    @pl.loop(0, 6)
    def _(j):
        pltpu.sync_copy(zb_v, acc_sh.at[pl.ds(sid * _RPS + j * 104, 104)])

    @pl.when(sid == 0)
    def _():
        pltpu.sync_copy(zb_v.at[pl.ds(0, 16)], acc_sh.at[pl.ds(16 * _RPS, 16)])

    tbl = hws_hbm.at[cid]
    _edge_pipeline(tbl, src_hbm, dst_hbm, ewn_hbm, sid * 2 * _WPW, 4,
                   sidx_v, didx_v, ew_v, rows, acc_sh, gsem, ssem)
    _edge_pipeline(tbl, src_hbm, dst_hbm, ewn_hbm, (sid * 2 + 1) * _WPW, 4,
                   sidx_v, didx_v, ew_v, rows, acc_sh, gsem, ssem)
    plsc.subcore_barrier()

    sl = pl.ds(sid * _RPS, _RPS)
    pltpu.async_copy(acc_sh.at[sl], out_hbm.at[cid, sl], g0).wait()

    @pl.when(sid == 0)
    def _():
        tl = pl.ds(16 * _RPS, 16)
        pltpu.async_copy(acc_sh.at[tl], out_hbm.at[cid, tl], g0).wait()


def _edge1_call(hws_ab, src, dst, ewn):
    k = pl.kernel(
        _edge1_kernel,
        out_type=jax.ShapeDtypeStruct((2, _N, 64), _f32),
        mesh=_MESH,
        compiler_params=_CP,
        scratch_types=[
            pltpu.VMEM((_WPW, _W), _i32),
            pltpu.VMEM((_WPW, _W), _i32),
            pltpu.VMEM((_WPW, _W), _f32),
            pltpu.VMEM((_W, 64), _f32),
            pltpu.VMEM((_W, 64), _f32),
            pltpu.VMEM((_W, 64), _f32),
            pltpu.VMEM((_W, 64), _f32),
            pltpu.VMEM((104, 64), _f32),
            pltpu.VMEM_SHARED((_N, 64), _f32),
        ] + [pltpu.SemaphoreType.DMA] * 8,
    )
    return k(hws_ab, src.reshape(_NWIN, _W), dst.reshape(_NWIN, _W),
             ewn.reshape(_NWIN, _W))


def kernel(x, edge_index, edge_attr, batch, W1, b1, W2, b2, W3, b3, Wlin, blin):
    pad = _EP - _E
    pad_idx = (jnp.arange(pad, dtype=_i32) * 64) % _N
    src = jnp.concatenate([edge_index[0], pad_idx])
    dst = jnp.concatenate([edge_index[1], pad_idx])
    ew = jnp.concatenate([edge_attr, jnp.zeros((pad,), _f32)])

    m, hw1 = pl.pallas_call(
        _pre_body,
        out_shape=(jax.ShapeDtypeStruct((8, 128), _f32),
                   jax.ShapeDtypeStruct((_N, 128), _f32)),
    )(edge_attr, x, W1)
    deg0, deg1 = _deg_call(dst, ew)

    dcol, hws1 = pl.pallas_call(
        _scale_split_body,
        out_shape=(jax.ShapeDtypeStruct((_N, 1), _f32),
                   jax.ShapeDtypeStruct((2, _N, 64), _f32)),
    )(hw1, deg0.reshape(_N, 1), deg1.reshape(_N, 1), m)

    acc1 = _edge1_call(hws1, src, dst, ew)
    hws2 = _tc(_layer1_body, (_N, 64))(acc1, hws1, dcol, m, b1, W2)
    a0, a1 = _edge_call(64, hws2, src, dst, ew)
    hws3 = _tc(_layer_body, (_N, 32))(a0, a1, hws2, dcol, m, b2, W3)
    a0, a1 = _edge_call(32, hws3, src, dst, ew)

    out = _tc(_final_body, (_NG, _OUT))(
        a0, a1, hws3, dcol, m, b3, batch.reshape(1, _N), Wlin, blin)
    return out.reshape(_NG, 75, 16)


# speedup vs baseline: 1.0500x; 1.0500x over previous
import dataclasses
import functools

import numpy as _np

import jax
import jax.numpy as jnp
from jax import lax
from jax.experimental import pallas as pl
from jax.experimental.pallas import tpu as pltpu
from jax.experimental.pallas import tpu_sc as plsc

_N = 10000
_E = 320000
_NG = 16
_OUT = 1200
_W = 128
_NWORK = 32
_WPW = 80
_NWIN = _NWORK * _WPW
_EP = _NWIN * _W
_RPS = 624

_f32 = jnp.float32
_i32 = jnp.int32


def _pre_body(ew_ref, x_ref, w_ref, m_ref, hw_ref):
    m_ref[...] = jnp.broadcast_to(jnp.max(ew_ref[...]), (8, 128))
    hw_ref[...] = jnp.dot(x_ref[...], w_ref[...], preferred_element_type=_f32)


def _ileave(h):
    f = h.shape[1]
    j = lax.broadcasted_iota(_i32, (f, f), 1)
    a = lax.broadcasted_iota(_i32, (f, f), 0)
    s = 32 * (j // 32) + 16 * (j % 2) + (j % 32) // 2
    perm = (a == s).astype(_f32)
    return jnp.dot(h, perm, preferred_element_type=_f32).astype(jnp.bfloat16)


def _scale_split_body(hw_ref, deg0c_ref, deg1c_ref, m_ref, dcol_ref, out_ref,
                      outb_ref):
    inv_m = 1.0 / m_ref[0, 0]
    deg = 1.0 + (deg0c_ref[...] + deg1c_ref[...]) * inv_m
    dcol = lax.rsqrt(deg)
    dcol_ref[...] = dcol
    hws = hw_ref[...] * dcol
    out_ref[0] = hws[:, :64]
    out_ref[1] = hws[:, 64:]
    outb_ref[0] = _ileave(hws[:, :64])
    outb_ref[1] = _ileave(hws[:, 64:])


def _layer1_body(acc_ref, hws_ref, dcol_ref, m_ref, b_ref, wn_ref, out_ref,
                 outb_ref):
    inv_m = 1.0 / m_ref[0, 0]
    dcol = dcol_ref[...]
    b = b_ref[...]
    agga = dcol * (acc_ref[0] * inv_m + hws_ref[0]) + b[:64]
    aggb = dcol * (acc_ref[1] * inv_m + hws_ref[1]) + b[64:]
    h = jnp.maximum(jnp.concatenate([agga, aggb], axis=1), 0.0)
    hws_n = jnp.dot(h, wn_ref[...], preferred_element_type=_f32) * dcol
    out_ref[...] = hws_n
    outb_ref[...] = _ileave(hws_n)


def _layer_body(acc0_ref, acc1_ref, hws_ref, dcol_ref, m_ref, b_ref, wn_ref,
                out_ref, outb_ref):
    inv_m = 1.0 / m_ref[0, 0]
    dcol = dcol_ref[...]
    agg = (dcol * ((acc0_ref[...] + acc1_ref[...]) * inv_m + hws_ref[...])
           + b_ref[...])
    h = jnp.maximum(agg, 0.0)
    hws_n = jnp.dot(h, wn_ref[...], preferred_element_type=_f32) * dcol
    out_ref[...] = hws_n
    outb_ref[...] = _ileave(hws_n)


def _final_body(acc0_ref, acc1_ref, hws_ref, dcol_ref, m_ref, b_ref, batch_ref,
                wlin_ref, blin_ref, out_ref):
    inv_m = 1.0 / m_ref[0, 0]
    agg = (dcol_ref[...] * ((acc0_ref[...] + acc1_ref[...]) * inv_m
                            + hws_ref[...]) + b_ref[...])
    h = jnp.maximum(agg, 0.0)
    gids = lax.broadcasted_iota(_i32, (_NG, _N), 0)
    oh = (batch_ref[...] == gids).astype(_f32)
    cnt = jnp.sum(oh, axis=1, keepdims=True)
    sums = jnp.dot(oh, h, preferred_element_type=_f32)
    pooled = sums / jnp.maximum(cnt, 1.0)
    out_ref[...] = (jnp.dot(pooled, wlin_ref[...], preferred_element_type=_f32)
                    + blin_ref[...])


def _tc(body, out_shape):
    return pl.pallas_call(body, out_shape=jax.ShapeDtypeStruct(out_shape, _f32))


_MESH = plsc.VectorSubcoreMesh(core_axis_name="c", subcore_axis_name="s")

_GDN = lax.GatherDimensionNumbers(
    offset_dims=(), collapsed_slice_dims=(0,), start_index_map=(0,),
    operand_batching_dims=(), start_indices_batching_dims=())

_CP = pltpu.CompilerParams()
if "needs_layout_passes" in pltpu.CompilerParams.__dataclass_fields__:
    _CP = dataclasses.replace(_CP, needs_layout_passes=False)
if "use_tc_tiling_on_sc" in pltpu.CompilerParams.__dataclass_fields__:
    _CP = dataclasses.replace(_CP, use_tc_tiling_on_sc=False)


def _deg_kernel(dst_hbm, ewn_hbm, out0_hbm, out1_hbm, didx_v, ew_v, zb_v,
                acc_sh, sem):
    cid = lax.axis_index("c")
    sid = lax.axis_index("s")
    wid = sid * 2 + cid

    z16 = jnp.zeros((16,), _f32)

    @pl.loop(0, 125)
    def _(r):
        zb_v[pl.ds(r * 16, 16)] = z16

    @pl.when(sid < 5)
    def _():
        pltpu.sync_copy(zb_v, acc_sh.at[pl.ds(sid * 2000, 2000)])

    pltpu.sync_copy(dst_hbm.at[pl.ds(wid * _WPW, _WPW)], didx_v)
    pltpu.sync_copy(ewn_hbm.at[pl.ds(wid * _WPW, _WPW)], ew_v)
    plsc.subcore_barrier()

    @pl.loop(0, _WPW, step=8)
    def _(w):
        for j in range(8):
            pltpu.async_copy(ew_v.at[w + j], acc_sh.at[didx_v.at[w + j]],
                             sem, add=True)
        for j in range(8):
            pltpu.make_async_copy(ewn_hbm.at[0], ew_v.at[0], sem).wait()

    plsc.subcore_barrier()

    @pl.when(sid < 5)
    def _():
        sl = pl.ds(sid * 2000, 2000)
        pltpu.sync_copy(acc_sh.at[sl], zb_v)

        @pl.when(cid == 0)
        def _():
            pltpu.sync_copy(zb_v, out0_hbm.at[sl])

        @pl.when(cid == 1)
        def _():
            pltpu.sync_copy(zb_v, out1_hbm.at[sl])


def _deg_call(dst, ewn):
    k = pl.kernel(
        _deg_kernel,
        out_type=(jax.ShapeDtypeStruct((_N,), _f32),
                  jax.ShapeDtypeStruct((_N,), _f32)),
        mesh=_MESH,
        compiler_params=_CP,
        scratch_types=[
            pltpu.VMEM((_WPW, _W), _i32),
            pltpu.VMEM((_WPW, _W), _f32),
            pltpu.VMEM((2000,), _f32),
            pltpu.VMEM_SHARED((_N,), _f32),
            pltpu.SemaphoreType.DMA,
        ],
    )
    return k(dst.reshape(_NWIN, _W), ewn.reshape(_NWIN, _W))


def _edge_pipeline(tbl_hbm, src_hbm, dst_hbm, ewn_hbm, woff, nc,
                   sidx_v, didx_v, ew_v, gbufs, rows, acc_sh, gsem, ssem):
    pltpu.sync_copy(src_hbm.at[pl.ds(woff, _WPW)], sidx_v)
    pltpu.sync_copy(dst_hbm.at[pl.ds(woff, _WPW)], didx_v)
    pltpu.sync_copy(ewn_hbm.at[pl.ds(woff, _WPW)], ew_v)

    def start_gather(w, j):
        pltpu.async_copy(tbl_hbm.at[sidx_v.at[w]], gbufs[j], gsem[j])

    def start_scatter(w, j):
        pltpu.async_copy(rows[j], acc_sh.at[didx_v.at[w]], ssem[j], add=True)

    def wait(ref, sem):
        pltpu.make_async_copy(tbl_hbm.at[pl.ds(0, _W)], ref, sem).wait()

    def wait_s(j):
        pltpu.make_async_copy(rows[j], acc_sh.at[didx_v.at[0]], ssem[j]).wait()

    def scale(w, j):
        @plsc.parallel_loop(0, _W, unroll=4)
        def _(r):
            g = plsc.load_gather(
                ew_v, [jnp.full((16,), w, _i32), jnp.full((16,), r, _i32)])
            for c in range(nc // 2):
                ab = gbufs[j][r, pl.ds(c * 32, 32)]
                a, b = plsc.unpack(ab, format=plsc.PackFormat.INTERLEAVED)
                rows[j][r, pl.ds(c * 32, 16)] = a * g
                rows[j][r, pl.ds(c * 32 + 16, 16)] = b * g

    for j in range(4):
        start_gather(j, j)

    @pl.loop(0, _WPW // 4 - 1)
    def _(p):
        w0 = p * 4
        wait(gbufs[0], gsem[0]); scale(w0 + 0, 0); start_scatter(w0 + 0, 0)
        wait(gbufs[1], gsem[1]); scale(w0 + 1, 1); start_scatter(w0 + 1, 1)
        wait_s(0); start_gather(w0 + 4, 0)
        wait(gbufs[2], gsem[2]); scale(w0 + 2, 2); start_scatter(w0 + 2, 2)
        wait_s(1); start_gather(w0 + 5, 1)
        wait(gbufs[3], gsem[3]); scale(w0 + 3, 3); start_scatter(w0 + 3, 3)
        wait_s(2); start_gather(w0 + 6, 2)
        wait_s(3); start_gather(w0 + 7, 3)

    wE = _WPW - 4
    wait(gbufs[0], gsem[0]); scale(wE + 0, 0); start_scatter(wE + 0, 0)
    wait(gbufs[1], gsem[1]); scale(wE + 1, 1); start_scatter(wE + 1, 1)
    wait_s(0)
    wait(gbufs[2], gsem[2]); scale(wE + 2, 2); start_scatter(wE + 2, 2)
    wait_s(1)
    wait(gbufs[3], gsem[3]); scale(wE + 3, 3); start_scatter(wE + 3, 3)
    wait_s(2)
    wait_s(3)


def _edge_kernel(F, hws_hbm, src_hbm, dst_hbm, ewn_hbm, out0_hbm, out1_hbm,
                 sidx_v, didx_v, ew_v, gb0, gb1, gb2, gb3, r0, r1, r2, r3,
                 zb_v, acc_sh, g0, g1, g2, g3, s0, s1, s2, s3):
    cid = lax.axis_index("c")
    sid = lax.axis_index("s")
    wid = sid * 2 + cid
    nc = F // 16
    gbufs = (gb0, gb1, gb2, gb3)
    rows = (r0, r1, r2, r3)
    gsem = (g0, g1, g2, g3)
    ssem = (s0, s1, s2, s3)

    z16 = jnp.zeros((16,), _f32)

    @pl.loop(0, 104)
    def _(r):
        for c in range(nc):
            zb_v[r, pl.ds(c * 16, 16)] = z16

    @pl.loop(0, 6)
    def _(j):
        pltpu.sync_copy(zb_v, acc_sh.at[pl.ds(sid * _RPS + j * 104, 104)])

    @pl.when(sid == 0)
    def _():
        pltpu.sync_copy(zb_v.at[pl.ds(0, 16)], acc_sh.at[pl.ds(16 * _RPS, 16)])

    _edge_pipeline(hws_hbm, src_hbm, dst_hbm, ewn_hbm, wid * _WPW, nc,
                   sidx_v, didx_v, ew_v, gbufs, rows, acc_sh, gsem, ssem)
    plsc.subcore_barrier()

    def _dump(out_hbm):
        sl = pl.ds(sid * _RPS, _RPS)
        pltpu.async_copy(acc_sh.at[sl], out_hbm.at[sl], g0).wait()

        @pl.when(sid == 0)
        def _():
            tl = pl.ds(16 * _RPS, 16)
            pltpu.async_copy(acc_sh.at[tl], out_hbm.at[tl], g0).wait()

    @pl.when(cid == 0)
    def _():
        _dump(out0_hbm)

    @pl.when(cid == 1)
    def _():
        _dump(out1_hbm)


def _edge_call(F, hws, src, dst, ewn):
    k = pl.kernel(
        functools.partial(_edge_kernel, F),
        out_type=(jax.ShapeDtypeStruct((_N, F), _f32),
                  jax.ShapeDtypeStruct((_N, F), _f32)),
        mesh=_MESH,
        compiler_params=_CP,
        scratch_types=[
            pltpu.VMEM((_WPW, _W), _i32),
            pltpu.VMEM((_WPW, _W), _i32),
            pltpu.VMEM((_WPW, _W), _f32),
            pltpu.VMEM((_W, F), jnp.bfloat16),
            pltpu.VMEM((_W, F), jnp.bfloat16),
            pltpu.VMEM((_W, F), jnp.bfloat16),
            pltpu.VMEM((_W, F), jnp.bfloat16),
            pltpu.VMEM((_W, F), _f32),
            pltpu.VMEM((_W, F), _f32),
            pltpu.VMEM((_W, F), _f32),
            pltpu.VMEM((_W, F), _f32),
            pltpu.VMEM((104, F), _f32),
            pltpu.VMEM_SHARED((_N, F), _f32),
        ] + [pltpu.SemaphoreType.DMA] * 8,
    )
    return k(hws, src.reshape(_NWIN, _W), dst.reshape(_NWIN, _W),
             ewn.reshape(_NWIN, _W))


def _edge1_kernel(hws_hbm, src_hbm, dst_hbm, ewn_hbm, out_hbm,
                  sidx_v, didx_v, ew_v, gb0, gb1, gb2, gb3, r0, r1, r2, r3,
                  zb_v, acc_sh, g0, g1, g2, g3, s0, s1, s2, s3):
    cid = lax.axis_index("c")
    sid = lax.axis_index("s")
    gbufs = (gb0, gb1, gb2, gb3)
    rows = (r0, r1, r2, r3)
    gsem = (g0, g1, g2, g3)
    ssem = (s0, s1, s2, s3)
    z16 = jnp.zeros((16,), _f32)

    @pl.loop(0, 104)
    def _(r):
        for c in range(4):
            zb_v[r, pl.ds(c * 16, 16)] = z16

    @pl.loop(0, 6)
    def _(j):
        pltpu.sync_copy(zb_v, acc_sh.at[pl.ds(sid * _RPS + j * 104, 104)])

    @pl.when(sid == 0)
    def _():
        pltpu.sync_copy(zb_v.at[pl.ds(0, 16)], acc_sh.at[pl.ds(16 * _RPS, 16)])

    tbl = hws_hbm.at[cid]
    _edge_pipeline(tbl, src_hbm, dst_hbm, ewn_hbm, sid * 2 * _WPW, 4,
                   sidx_v, didx_v, ew_v, gbufs, rows, acc_sh, gsem, ssem)
    _edge_pipeline(tbl, src_hbm, dst_hbm, ewn_hbm, (sid * 2 + 1) * _WPW, 4,
                   sidx_v, didx_v, ew_v, gbufs, rows, acc_sh, gsem, ssem)
    plsc.subcore_barrier()

    sl = pl.ds(sid * _RPS, _RPS)
    pltpu.async_copy(acc_sh.at[sl], out_hbm.at[cid, sl], g0).wait()

    @pl.when(sid == 0)
    def _():
        tl = pl.ds(16 * _RPS, 16)
        pltpu.async_copy(acc_sh.at[tl], out_hbm.at[cid, tl], g0).wait()


def _edge1_call(hws_ab, src, dst, ewn):
    k = pl.kernel(
        _edge1_kernel,
        out_type=jax.ShapeDtypeStruct((2, _N, 64), _f32),
        mesh=_MESH,
        compiler_params=_CP,
        scratch_types=[
            pltpu.VMEM((_WPW, _W), _i32),
            pltpu.VMEM((_WPW, _W), _i32),
            pltpu.VMEM((_WPW, _W), _f32),
            pltpu.VMEM((_W, 64), jnp.bfloat16),
            pltpu.VMEM((_W, 64), jnp.bfloat16),
            pltpu.VMEM((_W, 64), jnp.bfloat16),
            pltpu.VMEM((_W, 64), jnp.bfloat16),
            pltpu.VMEM((_W, 64), _f32),
            pltpu.VMEM((_W, 64), _f32),
            pltpu.VMEM((_W, 64), _f32),
            pltpu.VMEM((_W, 64), _f32),
            pltpu.VMEM((104, 64), _f32),
            pltpu.VMEM_SHARED((_N, 64), _f32),
        ] + [pltpu.SemaphoreType.DMA] * 8,
    )
    return k(hws_ab, src.reshape(_NWIN, _W), dst.reshape(_NWIN, _W),
             ewn.reshape(_NWIN, _W))


def kernel(x, edge_index, edge_attr, batch, W1, b1, W2, b2, W3, b3, Wlin, blin):
    pad = _EP - _E
    pad_idx = (jnp.arange(pad, dtype=_i32) * 64) % _N
    src = jnp.concatenate([edge_index[0], pad_idx])
    dst = jnp.concatenate([edge_index[1], pad_idx])
    ew = jnp.concatenate([edge_attr, jnp.zeros((pad,), _f32)])

    m, hw1 = pl.pallas_call(
        _pre_body,
        out_shape=(jax.ShapeDtypeStruct((8, 128), _f32),
                   jax.ShapeDtypeStruct((_N, 128), _f32)),
    )(edge_attr, x, W1)
    deg0, deg1 = _deg_call(dst, ew)

    dcol, hws1, hws1b = pl.pallas_call(
        _scale_split_body,
        out_shape=(jax.ShapeDtypeStruct((_N, 1), _f32),
                   jax.ShapeDtypeStruct((2, _N, 64), _f32),
                   jax.ShapeDtypeStruct((2, _N, 64), jnp.bfloat16)),
    )(hw1, deg0.reshape(_N, 1), deg1.reshape(_N, 1), m)

    acc1 = _edge1_call(hws1b, src, dst, ew)
    hws2, hws2b = pl.pallas_call(
        _layer1_body,
        out_shape=(jax.ShapeDtypeStruct((_N, 64), _f32),
                   jax.ShapeDtypeStruct((_N, 64), jnp.bfloat16)),
    )(acc1, hws1, dcol, m, b1, W2)
    a0, a1 = _edge_call(64, hws2b, src, dst, ew)
    hws3, hws3b = pl.pallas_call(
        _layer_body,
        out_shape=(jax.ShapeDtypeStruct((_N, 32), _f32),
                   jax.ShapeDtypeStruct((_N, 32), jnp.bfloat16)),
    )(a0, a1, hws2, dcol, m, b2, W3)
    a0, a1 = _edge_call(32, hws3b, src, dst, ew)

    out = _tc(_final_body, (_NG, _OUT))(
        a0, a1, hws3, dcol, m, b3, batch.reshape(1, _N), Wlin, blin)
    return out.reshape(_NG, 75, 16)


# speedup vs baseline: 1.0722x; 1.0211x over previous
import dataclasses
import functools

import numpy as _np

import jax
import jax.numpy as jnp
from jax import lax
from jax.experimental import pallas as pl
from jax.experimental.pallas import tpu as pltpu
from jax.experimental.pallas import tpu_sc as plsc

_N = 10000
_E = 320000
_NG = 16
_OUT = 1200
_W = 128
_NWORK = 32
_WPW = 80
_NWIN = _NWORK * _WPW
_EP = _NWIN * _W
_RPS = 624

_f32 = jnp.float32
_i32 = jnp.int32


def _pre_body(ew_ref, x_ref, w_ref, m_ref, hw_ref, hwb_ref):
    m_ref[...] = jnp.broadcast_to(jnp.max(ew_ref[...]), (8, 128))
    hw = jnp.dot(x_ref[...], w_ref[...], preferred_element_type=_f32)
    hw_ref[...] = hw
    hwb_ref[0] = _ileave(hw[:, :64])
    hwb_ref[1] = _ileave(hw[:, 64:])


def _ileave(h):
    f = h.shape[1]
    j = lax.broadcasted_iota(_i32, (f, f), 1)
    a = lax.broadcasted_iota(_i32, (f, f), 0)
    s = 32 * (j // 32) + 16 * (j % 2) + (j % 32) // 2
    perm = (a == s).astype(_f32)
    return jnp.dot(h, perm, preferred_element_type=_f32).astype(jnp.bfloat16)


def _layer1_body(acc_ref, hw1_ref, dcol_ref, m_ref, b_ref, wn_ref, out_ref,
                 outb_ref):
    inv_m = 1.0 / m_ref[0, 0]
    dcol = dcol_ref[...]
    b = b_ref[...]
    hws1 = hw1_ref[...] * dcol
    agga = dcol * (acc_ref[0] * inv_m + hws1[:, :64]) + b[:64]
    aggb = dcol * (acc_ref[1] * inv_m + hws1[:, 64:]) + b[64:]
    h = jnp.maximum(jnp.concatenate([agga, aggb], axis=1), 0.0)
    hws_n = jnp.dot(h, wn_ref[...], preferred_element_type=_f32) * dcol
    out_ref[...] = hws_n
    outb_ref[...] = _ileave(hws_n)


def _layer_body(acc0_ref, acc1_ref, hws_ref, dcol_ref, m_ref, b_ref, wn_ref,
                out_ref, outb_ref):
    inv_m = 1.0 / m_ref[0, 0]
    dcol = dcol_ref[...]
    agg = (dcol * ((acc0_ref[...] + acc1_ref[...]) * inv_m + hws_ref[...])
           + b_ref[...])
    h = jnp.maximum(agg, 0.0)
    hws_n = jnp.dot(h, wn_ref[...], preferred_element_type=_f32) * dcol
    out_ref[...] = hws_n
    outb_ref[...] = _ileave(hws_n)


def _final_body(acc0_ref, acc1_ref, hws_ref, dcol_ref, m_ref, b_ref, batch_ref,
                wlin_ref, blin_ref, out_ref):
    inv_m = 1.0 / m_ref[0, 0]
    agg = (dcol_ref[...] * ((acc0_ref[...] + acc1_ref[...]) * inv_m
                            + hws_ref[...]) + b_ref[...])
    h = jnp.maximum(agg, 0.0)
    gids = lax.broadcasted_iota(_i32, (_NG, _N), 0)
    oh = (batch_ref[...] == gids).astype(_f32)
    cnt = jnp.sum(oh, axis=1, keepdims=True)
    sums = jnp.dot(oh, h, preferred_element_type=_f32)
    pooled = sums / jnp.maximum(cnt, 1.0)
    out_ref[...] = (jnp.dot(pooled, wlin_ref[...], preferred_element_type=_f32)
                    + blin_ref[...])


def _tc(body, out_shape):
    return pl.pallas_call(body, out_shape=jax.ShapeDtypeStruct(out_shape, _f32))


_MESH = plsc.VectorSubcoreMesh(core_axis_name="c", subcore_axis_name="s")

_GDN = lax.GatherDimensionNumbers(
    offset_dims=(), collapsed_slice_dims=(0,), start_index_map=(0,),
    operand_batching_dims=(), start_indices_batching_dims=())

_CP = pltpu.CompilerParams()
if "needs_layout_passes" in pltpu.CompilerParams.__dataclass_fields__:
    _CP = dataclasses.replace(_CP, needs_layout_passes=False)
if "use_tc_tiling_on_sc" in pltpu.CompilerParams.__dataclass_fields__:
    _CP = dataclasses.replace(_CP, use_tc_tiling_on_sc=False)


def _deg_kernel(dst_hbm, ewn_hbm, out0_hbm, out1_hbm, didx_v, ew_v, zb_v,
                acc_sh, sem):
    cid = lax.axis_index("c")
    sid = lax.axis_index("s")
    wid = sid * 2 + cid

    z16 = jnp.zeros((16,), _f32)

    @pl.loop(0, 125)
    def _(r):
        zb_v[pl.ds(r * 16, 16)] = z16

    @pl.when(sid < 5)
    def _():
        pltpu.sync_copy(zb_v, acc_sh.at[pl.ds(sid * 2000, 2000)])

    pltpu.sync_copy(dst_hbm.at[pl.ds(wid * _WPW, _WPW)], didx_v)
    pltpu.sync_copy(ewn_hbm.at[pl.ds(wid * _WPW, _WPW)], ew_v)
    plsc.subcore_barrier()

    @pl.loop(0, _WPW, step=8)
    def _(w):
        for j in range(8):
            pltpu.async_copy(ew_v.at[w + j], acc_sh.at[didx_v.at[w + j]],
                             sem, add=True)
        for j in range(8):
            pltpu.make_async_copy(ewn_hbm.at[0], ew_v.at[0], sem).wait()

    plsc.subcore_barrier()

    @pl.when(sid < 5)
    def _():
        sl = pl.ds(sid * 2000, 2000)
        pltpu.sync_copy(acc_sh.at[sl], zb_v)

        @pl.when(cid == 0)
        def _():
            pltpu.sync_copy(zb_v, out0_hbm.at[sl])

        @pl.when(cid == 1)
        def _():
            pltpu.sync_copy(zb_v, out1_hbm.at[sl])


def _deg_call(dst, ewn):
    k = pl.kernel(
        _deg_kernel,
        out_type=(jax.ShapeDtypeStruct((_N,), _f32),
                  jax.ShapeDtypeStruct((_N,), _f32)),
        mesh=_MESH,
        compiler_params=_CP,
        scratch_types=[
            pltpu.VMEM((_WPW, _W), _i32),
            pltpu.VMEM((_WPW, _W), _f32),
            pltpu.VMEM((2000,), _f32),
            pltpu.VMEM_SHARED((_N,), _f32),
            pltpu.SemaphoreType.DMA,
        ],
    )
    return k(dst.reshape(_NWIN, _W), ewn.reshape(_NWIN, _W))


def _edge_pipeline(tbl_hbm, src_hbm, dst_hbm, ewn_hbm, woff, nc,
                   sidx_v, didx_v, ew_v, gbufs, rows, acc_sh, gsem, ssem,
                   dinv_v=None):
    pltpu.sync_copy(src_hbm.at[pl.ds(woff, _WPW)], sidx_v)
    pltpu.sync_copy(dst_hbm.at[pl.ds(woff, _WPW)], didx_v)
    pltpu.sync_copy(ewn_hbm.at[pl.ds(woff, _WPW)], ew_v)

    def start_gather(w, j):
        pltpu.async_copy(tbl_hbm.at[sidx_v.at[w]], gbufs[j], gsem[j])

    def start_scatter(w, j):
        pltpu.async_copy(rows[j], acc_sh.at[didx_v.at[w]], ssem[j], add=True)

    def wait(ref, sem):
        pltpu.make_async_copy(tbl_hbm.at[pl.ds(0, _W)], ref, sem).wait()

    def wait_s(j):
        pltpu.make_async_copy(rows[j], acc_sh.at[didx_v.at[0]], ssem[j]).wait()

    def scale(w, j):
        @plsc.parallel_loop(0, _W, unroll=4)
        def _(r):
            g = plsc.load_gather(
                ew_v, [jnp.full((16,), w, _i32), jnp.full((16,), r, _i32)])
            if dinv_v is not None:
                s16 = plsc.load_gather(
                    sidx_v, [jnp.full((16,), w, _i32), jnp.full((16,), r, _i32)])
                g = g * plsc.load_gather(dinv_v, [s16])
            for c in range(nc // 2):
                ab = gbufs[j][r, pl.ds(c * 32, 32)]
                a, b = plsc.unpack(ab, format=plsc.PackFormat.INTERLEAVED)
                rows[j][r, pl.ds(c * 32, 16)] = a * g
                rows[j][r, pl.ds(c * 32 + 16, 16)] = b * g

    for j in range(4):
        start_gather(j, j)

    @pl.loop(0, _WPW // 4 - 1)
    def _(p):
        w0 = p * 4
        wait(gbufs[0], gsem[0]); scale(w0 + 0, 0); start_scatter(w0 + 0, 0)
        wait(gbufs[1], gsem[1]); scale(w0 + 1, 1); start_scatter(w0 + 1, 1)
        wait_s(0); start_gather(w0 + 4, 0)
        wait(gbufs[2], gsem[2]); scale(w0 + 2, 2); start_scatter(w0 + 2, 2)
        wait_s(1); start_gather(w0 + 5, 1)
        wait(gbufs[3], gsem[3]); scale(w0 + 3, 3); start_scatter(w0 + 3, 3)
        wait_s(2); start_gather(w0 + 6, 2)
        wait_s(3); start_gather(w0 + 7, 3)

    wE = _WPW - 4
    wait(gbufs[0], gsem[0]); scale(wE + 0, 0); start_scatter(wE + 0, 0)
    wait(gbufs[1], gsem[1]); scale(wE + 1, 1); start_scatter(wE + 1, 1)
    wait_s(0)
    wait(gbufs[2], gsem[2]); scale(wE + 2, 2); start_scatter(wE + 2, 2)
    wait_s(1)
    wait(gbufs[3], gsem[3]); scale(wE + 3, 3); start_scatter(wE + 3, 3)
    wait_s(2)
    wait_s(3)


def _edge_kernel(F, hws_hbm, src_hbm, dst_hbm, ewn_hbm, out0_hbm, out1_hbm,
                 sidx_v, didx_v, ew_v, gb0, gb1, gb2, gb3, r0, r1, r2, r3,
                 zb_v, acc_sh, g0, g1, g2, g3, s0, s1, s2, s3):
    cid = lax.axis_index("c")
    sid = lax.axis_index("s")
    wid = sid * 2 + cid
    nc = F // 16
    gbufs = (gb0, gb1, gb2, gb3)
    rows = (r0, r1, r2, r3)
    gsem = (g0, g1, g2, g3)
    ssem = (s0, s1, s2, s3)

    z16 = jnp.zeros((16,), _f32)

    @pl.loop(0, 104)
    def _(r):
        for c in range(nc):
            zb_v[r, pl.ds(c * 16, 16)] = z16

    @pl.loop(0, 6)
    def _(j):
        pltpu.sync_copy(zb_v, acc_sh.at[pl.ds(sid * _RPS + j * 104, 104)])

    @pl.when(sid == 0)
    def _():
        pltpu.sync_copy(zb_v.at[pl.ds(0, 16)], acc_sh.at[pl.ds(16 * _RPS, 16)])

    _edge_pipeline(hws_hbm, src_hbm, dst_hbm, ewn_hbm, wid * _WPW, nc,
                   sidx_v, didx_v, ew_v, gbufs, rows, acc_sh, gsem, ssem)
    plsc.subcore_barrier()

    def _dump(out_hbm):
        sl = pl.ds(sid * _RPS, _RPS)
        pltpu.async_copy(acc_sh.at[sl], out_hbm.at[sl], g0).wait()

        @pl.when(sid == 0)
        def _():
            tl = pl.ds(16 * _RPS, 16)
            pltpu.async_copy(acc_sh.at[tl], out_hbm.at[tl], g0).wait()

    @pl.when(cid == 0)
    def _():
        _dump(out0_hbm)

    @pl.when(cid == 1)
    def _():
        _dump(out1_hbm)


def _edge_call(F, hws, src, dst, ewn):
    k = pl.kernel(
        functools.partial(_edge_kernel, F),
        out_type=(jax.ShapeDtypeStruct((_N, F), _f32),
                  jax.ShapeDtypeStruct((_N, F), _f32)),
        mesh=_MESH,
        compiler_params=_CP,
        scratch_types=[
            pltpu.VMEM((_WPW, _W), _i32),
            pltpu.VMEM((_WPW, _W), _i32),
            pltpu.VMEM((_WPW, _W), _f32),
            pltpu.VMEM((_W, F), jnp.bfloat16),
            pltpu.VMEM((_W, F), jnp.bfloat16),
            pltpu.VMEM((_W, F), jnp.bfloat16),
            pltpu.VMEM((_W, F), jnp.bfloat16),
            pltpu.VMEM((_W, F), _f32),
            pltpu.VMEM((_W, F), _f32),
            pltpu.VMEM((_W, F), _f32),
            pltpu.VMEM((_W, F), _f32),
            pltpu.VMEM((104, F), _f32),
            pltpu.VMEM_SHARED((_N, F), _f32),
        ] + [pltpu.SemaphoreType.DMA] * 8,
    )
    return k(hws, src.reshape(_NWIN, _W), dst.reshape(_NWIN, _W),
             ewn.reshape(_NWIN, _W))


def _edge1_kernel(hws_hbm, src_hbm, dst_hbm, ewn_hbm, m_hbm, out_hbm,
                  odinv_hbm, sidx_v, didx_v, ew_v, gb0, gb1, gb2, gb3,
                  r0, r1, r2, r3, dinv_v, m_v, acc_sh, deg_sh,
                  g0, g1, g2, g3, s0, s1, s2, s3):
    cid = lax.axis_index("c")
    sid = lax.axis_index("s")
    gbufs = (gb0, gb1, gb2, gb3)
    rows = (r0, r1, r2, r3)
    gsem = (g0, g1, g2, g3)
    ssem = (s0, s1, s2, s3)
    z16 = jnp.zeros((16,), _f32)

    pltpu.sync_copy(m_hbm.at[0, pl.ds(0, 16)], m_v)

    @pl.loop(0, _W)
    def _(r):
        for buf in rows:
            for c in range(4):
                buf[r, pl.ds(c * 16, 16)] = z16

    for q in range(4):
        pltpu.sync_copy(rows[q], acc_sh.at[pl.ds(sid * _RPS + q * _W, _W)])
    pltpu.sync_copy(r0.at[pl.ds(0, 112)],
                    acc_sh.at[pl.ds(sid * _RPS + 4 * _W, 112)])

    @pl.when(sid == 0)
    def _():
        pltpu.sync_copy(r0.at[pl.ds(0, 16)], acc_sh.at[pl.ds(16 * _RPS, 16)])

    @pl.when(sid < 5)
    def _():
        @pl.loop(0, 125)
        def _(r):
            dinv_v[pl.ds(r * 16, 16)] = z16
        pltpu.sync_copy(dinv_v.at[pl.ds(0, 2000)],
                        deg_sh.at[pl.ds(sid * 2000, 2000)])
    plsc.subcore_barrier()

    for ph in range(2):
        woff = (sid * 2 + ph) * _WPW
        pltpu.sync_copy(dst_hbm.at[pl.ds(woff, _WPW)], didx_v)
        pltpu.sync_copy(ewn_hbm.at[pl.ds(woff, _WPW)], ew_v)

        @pl.loop(0, _WPW, step=8)
        def _(w):
            for j in range(8):
                pltpu.async_copy(ew_v.at[w + j], deg_sh.at[didx_v.at[w + j]],
                                 g0, add=True)
            for j in range(8):
                pltpu.make_async_copy(ewn_hbm.at[0], ew_v.at[0], g0).wait()

    plsc.subcore_barrier()

    @pl.when(sid < 5)
    def _():
        sl = pl.ds(sid * 2000, 2000)
        pltpu.sync_copy(deg_sh.at[sl], dinv_v.at[pl.ds(0, 2000)])
        inv_m = 1.0 / m_v[...]
        magic = jnp.full((16,), 0x5F3759DF, _i32)

        @pl.loop(0, 125)
        def _(i):
            d = 1.0 + dinv_v[pl.ds(i * 16, 16)] * inv_m
            y = plsc.bitcast(
                magic - lax.shift_right_logical(plsc.bitcast(d, _i32), 1),
                _f32)
            for _it in range(3):
                y = y * (1.5 - 0.5 * d * y * y)
            dinv_v[pl.ds(i * 16, 16)] = y

        pltpu.sync_copy(dinv_v.at[pl.ds(0, 2000)], deg_sh.at[sl])
    plsc.subcore_barrier()

    pltpu.sync_copy(deg_sh, dinv_v)

    @pl.when((sid < 5) & (cid == 0))
    def _():
        sl = pl.ds(sid * 2000, 2000)
        pltpu.sync_copy(dinv_v.at[sl], odinv_hbm.at[sl])

    tbl = hws_hbm.at[cid]
    _edge_pipeline(tbl, src_hbm, dst_hbm, ewn_hbm, sid * 2 * _WPW, 4,
                   sidx_v, didx_v, ew_v, gbufs, rows, acc_sh, gsem, ssem,
                   dinv_v=dinv_v)
    _edge_pipeline(tbl, src_hbm, dst_hbm, ewn_hbm, (sid * 2 + 1) * _WPW, 4,
                   sidx_v, didx_v, ew_v, gbufs, rows, acc_sh, gsem, ssem,
                   dinv_v=dinv_v)
    plsc.subcore_barrier()

    sl = pl.ds(sid * _RPS, _RPS)
    pltpu.async_copy(acc_sh.at[sl], out_hbm.at[cid, sl], g0).wait()

    @pl.when(sid == 0)
    def _():
        tl = pl.ds(16 * _RPS, 16)
        pltpu.async_copy(acc_sh.at[tl], out_hbm.at[cid, tl], g0).wait()


def _edge1_call(hws_ab, src, dst, ewn, m):
    k = pl.kernel(
        _edge1_kernel,
        out_type=(jax.ShapeDtypeStruct((2, _N, 64), _f32),
                  jax.ShapeDtypeStruct((_N,), _f32)),
        mesh=_MESH,
        compiler_params=_CP,
        scratch_types=[
            pltpu.VMEM((_WPW, _W), _i32),
            pltpu.VMEM((_WPW, _W), _i32),
            pltpu.VMEM((_WPW, _W), _f32),
            pltpu.VMEM((_W, 64), jnp.bfloat16),
            pltpu.VMEM((_W, 64), jnp.bfloat16),
            pltpu.VMEM((_W, 64), jnp.bfloat16),
            pltpu.VMEM((_W, 64), jnp.bfloat16),
            pltpu.VMEM((_W, 64), _f32),
            pltpu.VMEM((_W, 64), _f32),
            pltpu.VMEM((_W, 64), _f32),
            pltpu.VMEM((_W, 64), _f32),
            pltpu.VMEM((_N,), _f32),
            pltpu.VMEM((16,), _f32),
            pltpu.VMEM_SHARED((_N, 64), _f32),
            pltpu.VMEM_SHARED((_N,), _f32),
        ] + [pltpu.SemaphoreType.DMA] * 8,
    )
    return k(hws_ab, src.reshape(_NWIN, _W), dst.reshape(_NWIN, _W),
             ewn.reshape(_NWIN, _W), m)


def kernel(x, edge_index, edge_attr, batch, W1, b1, W2, b2, W3, b3, Wlin, blin):
    pad = _EP - _E
    pad_idx = (jnp.arange(pad, dtype=_i32) * 64) % _N
    src = jnp.concatenate([edge_index[0], pad_idx])
    dst = jnp.concatenate([edge_index[1], pad_idx])
    ew = jnp.concatenate([edge_attr, jnp.zeros((pad,), _f32)])

    m, hw1, hw1b = pl.pallas_call(
        _pre_body,
        out_shape=(jax.ShapeDtypeStruct((8, 128), _f32),
                   jax.ShapeDtypeStruct((_N, 128), _f32),
                   jax.ShapeDtypeStruct((2, _N, 64), jnp.bfloat16)),
    )(edge_attr, x, W1)

    acc1, dinv = _edge1_call(hw1b, src, dst, ew, m)
    dcol = dinv.reshape(_N, 1)
    hws2, hws2b = pl.pallas_call(
        _layer1_body,
        out_shape=(jax.ShapeDtypeStruct((_N, 64), _f32),
                   jax.ShapeDtypeStruct((_N, 64), jnp.bfloat16)),
    )(acc1, hw1, dcol, m, b1, W2)
    a0, a1 = _edge_call(64, hws2b, src, dst, ew)
    hws3, hws3b = pl.pallas_call(
        _layer_body,
        out_shape=(jax.ShapeDtypeStruct((_N, 32), _f32),
                   jax.ShapeDtypeStruct((_N, 32), jnp.bfloat16)),
    )(a0, a1, hws2, dcol, m, b2, W3)
    a0, a1 = _edge_call(32, hws3b, src, dst, ew)

    out = _tc(_final_body, (_NG, _OUT))(
        a0, a1, hws3, dcol, m, b3, batch.reshape(1, _N), Wlin, blin)
    return out.reshape(_NG, 75, 16)


# speedup vs baseline: 1.0726x; 1.0004x over previous
import dataclasses
import functools


import jax
import jax.numpy as jnp
from jax import lax
from jax.experimental import pallas as pl
from jax.experimental.pallas import tpu as pltpu
from jax.experimental.pallas import tpu_sc as plsc

_N = 10000
_E = 320000
_NG = 16
_OUT = 1200
_W = 128
_NWORK = 32
_WPW = 80
_NWIN = _NWORK * _WPW
_EP = _NWIN * _W
_RPS = 624

_f32 = jnp.float32
_i32 = jnp.int32


def _pre_body(ew_ref, x_ref, w_ref, m_ref, hw_ref, hwb_ref):
    m_ref[...] = jnp.broadcast_to(jnp.max(ew_ref[...]), (8, 128))
    hw = jnp.dot(x_ref[...], w_ref[...], preferred_element_type=_f32)
    hw_ref[...] = hw
    hwb_ref[0] = _ileave(hw[:, :64])
    hwb_ref[1] = _ileave(hw[:, 64:])


def _ileave(h):
    f = h.shape[1]
    j = lax.broadcasted_iota(_i32, (f, f), 1)
    a = lax.broadcasted_iota(_i32, (f, f), 0)
    s = 32 * (j // 32) + 16 * (j % 2) + (j % 32) // 2
    perm = (a == s).astype(_f32)
    return jnp.dot(h, perm, preferred_element_type=_f32).astype(jnp.bfloat16)


def _layer1_body(acc_ref, hw1_ref, dcol_ref, m_ref, b_ref, wn_ref, out_ref,
                 outb_ref):
    inv_m = 1.0 / m_ref[0, 0]
    dcol = dcol_ref[...]
    b = b_ref[...]
    hws1 = hw1_ref[...] * dcol
    agga = dcol * (acc_ref[0] * inv_m + hws1[:, :64]) + b[:64]
    aggb = dcol * (acc_ref[1] * inv_m + hws1[:, 64:]) + b[64:]
    h = jnp.maximum(jnp.concatenate([agga, aggb], axis=1), 0.0)
    hws_n = jnp.dot(h, wn_ref[...], preferred_element_type=_f32) * dcol
    out_ref[...] = hws_n
    outb_ref[...] = _ileave(hws_n)


def _layer_body(acc0_ref, acc1_ref, hws_ref, dcol_ref, m_ref, b_ref, wn_ref,
                out_ref, outb_ref):
    inv_m = 1.0 / m_ref[0, 0]
    dcol = dcol_ref[...]
    agg = (dcol * ((acc0_ref[...] + acc1_ref[...]) * inv_m + hws_ref[...])
           + b_ref[...])
    h = jnp.maximum(agg, 0.0)
    hws_n = jnp.dot(h, wn_ref[...], preferred_element_type=_f32) * dcol
    out_ref[...] = hws_n
    outb_ref[...] = _ileave(hws_n)


def _final_body(acc0_ref, acc1_ref, hws_ref, dcol_ref, m_ref, b_ref, batch_ref,
                wlin_ref, blin_ref, out_ref):
    inv_m = 1.0 / m_ref[0, 0]
    agg = (dcol_ref[...] * ((acc0_ref[...] + acc1_ref[...]) * inv_m
                            + hws_ref[...]) + b_ref[...])
    h = jnp.maximum(agg, 0.0)
    gids = lax.broadcasted_iota(_i32, (_NG, _N), 0)
    oh = (batch_ref[...] == gids).astype(_f32)
    cnt = jnp.sum(oh, axis=1, keepdims=True)
    sums = jnp.dot(oh, h, preferred_element_type=_f32)
    pooled = sums / jnp.maximum(cnt, 1.0)
    out_ref[...] = (jnp.dot(pooled, wlin_ref[...], preferred_element_type=_f32)
                    + blin_ref[...])


def _tc(body, out_shape):
    return pl.pallas_call(body, out_shape=jax.ShapeDtypeStruct(out_shape, _f32))


_MESH = plsc.VectorSubcoreMesh(core_axis_name="c", subcore_axis_name="s")

_CP = pltpu.CompilerParams()
if "needs_layout_passes" in pltpu.CompilerParams.__dataclass_fields__:
    _CP = dataclasses.replace(_CP, needs_layout_passes=False)
if "use_tc_tiling_on_sc" in pltpu.CompilerParams.__dataclass_fields__:
    _CP = dataclasses.replace(_CP, use_tc_tiling_on_sc=False)


def _edge_pipeline(tbl_hbm, src_hbm, dst_hbm, ewn_hbm, woff, nc,
                   sidx_v, didx_v, ew_v, gbufs, rows, acc_sh, gsem, ssem,
                   dinv_v=None):
    pltpu.sync_copy(src_hbm.at[pl.ds(woff, _WPW)], sidx_v)
    pltpu.sync_copy(dst_hbm.at[pl.ds(woff, _WPW)], didx_v)
    pltpu.sync_copy(ewn_hbm.at[pl.ds(woff, _WPW)], ew_v)

    def start_gather(w, j):
        pltpu.async_copy(tbl_hbm.at[sidx_v.at[w]], gbufs[j], gsem[j])

    def start_scatter(w, j):
        pltpu.async_copy(rows[j], acc_sh.at[didx_v.at[w]], ssem[j], add=True)

    def wait(ref, sem):
        pltpu.make_async_copy(tbl_hbm.at[pl.ds(0, _W)], ref, sem).wait()

    def wait_s(j):
        pltpu.make_async_copy(rows[j], acc_sh.at[didx_v.at[0]], ssem[j]).wait()

    def scale(w, j):
        @plsc.parallel_loop(0, _W, unroll=4)
        def _(r):
            g = plsc.load_gather(
                ew_v, [jnp.full((16,), w, _i32), jnp.full((16,), r, _i32)])
            if dinv_v is not None:
                s16 = plsc.load_gather(
                    sidx_v, [jnp.full((16,), w, _i32), jnp.full((16,), r, _i32)])
                g = g * plsc.load_gather(dinv_v, [s16])
            for c in range(nc // 2):
                ab = gbufs[j][r, pl.ds(c * 32, 32)]
                a, b = plsc.unpack(ab, format=plsc.PackFormat.INTERLEAVED)
                rows[j][r, pl.ds(c * 32, 16)] = a * g
                rows[j][r, pl.ds(c * 32 + 16, 16)] = b * g

    for j in range(4):
        start_gather(j, j)

    @pl.loop(0, _WPW // 4 - 1)
    def _(p):
        w0 = p * 4
        wait(gbufs[0], gsem[0]); scale(w0 + 0, 0); start_scatter(w0 + 0, 0)
        wait(gbufs[1], gsem[1]); scale(w0 + 1, 1); start_scatter(w0 + 1, 1)
        wait_s(0); start_gather(w0 + 4, 0)
        wait(gbufs[2], gsem[2]); scale(w0 + 2, 2); start_scatter(w0 + 2, 2)
        wait_s(1); start_gather(w0 + 5, 1)
        wait(gbufs[3], gsem[3]); scale(w0 + 3, 3); start_scatter(w0 + 3, 3)
        wait_s(2); start_gather(w0 + 6, 2)
        wait_s(3); start_gather(w0 + 7, 3)

    wE = _WPW - 4
    wait(gbufs[0], gsem[0]); scale(wE + 0, 0); start_scatter(wE + 0, 0)
    wait(gbufs[1], gsem[1]); scale(wE + 1, 1); start_scatter(wE + 1, 1)
    wait_s(0)
    wait(gbufs[2], gsem[2]); scale(wE + 2, 2); start_scatter(wE + 2, 2)
    wait_s(1)
    wait(gbufs[3], gsem[3]); scale(wE + 3, 3); start_scatter(wE + 3, 3)
    wait_s(2)
    wait_s(3)


def _edge_kernel(F, hws_hbm, src_hbm, dst_hbm, ewn_hbm, out0_hbm, out1_hbm,
                 sidx_v, didx_v, ew_v, gb0, gb1, gb2, gb3, r0, r1, r2, r3,
                 zb_v, acc_sh, g0, g1, g2, g3, s0, s1, s2, s3):
    cid = lax.axis_index("c")
    sid = lax.axis_index("s")
    wid = sid * 2 + cid
    nc = F // 16
    gbufs = (gb0, gb1, gb2, gb3)
    rows = (r0, r1, r2, r3)
    gsem = (g0, g1, g2, g3)
    ssem = (s0, s1, s2, s3)

    z16 = jnp.zeros((16,), _f32)

    @pl.loop(0, 104)
    def _(r):
        for c in range(nc):
            zb_v[r, pl.ds(c * 16, 16)] = z16

    @pl.loop(0, 6)
    def _(j):
        pltpu.sync_copy(zb_v, acc_sh.at[pl.ds(sid * _RPS + j * 104, 104)])

    @pl.when(sid == 0)
    def _():
        pltpu.sync_copy(zb_v.at[pl.ds(0, 16)], acc_sh.at[pl.ds(16 * _RPS, 16)])

    _edge_pipeline(hws_hbm, src_hbm, dst_hbm, ewn_hbm, wid * _WPW, nc,
                   sidx_v, didx_v, ew_v, gbufs, rows, acc_sh, gsem, ssem)
    plsc.subcore_barrier()

    def _dump(out_hbm):
        sl = pl.ds(sid * _RPS, _RPS)
        pltpu.async_copy(acc_sh.at[sl], out_hbm.at[sl], g0).wait()

        @pl.when(sid == 0)
        def _():
            tl = pl.ds(16 * _RPS, 16)
            pltpu.async_copy(acc_sh.at[tl], out_hbm.at[tl], g0).wait()

    @pl.when(cid == 0)
    def _():
        _dump(out0_hbm)

    @pl.when(cid == 1)
    def _():
        _dump(out1_hbm)


def _edge_call(F, hws, src, dst, ewn):
    k = pl.kernel(
        functools.partial(_edge_kernel, F),
        out_type=(jax.ShapeDtypeStruct((_N, F), _f32),
                  jax.ShapeDtypeStruct((_N, F), _f32)),
        mesh=_MESH,
        compiler_params=_CP,
        scratch_types=[
            pltpu.VMEM((_WPW, _W), _i32),
            pltpu.VMEM((_WPW, _W), _i32),
            pltpu.VMEM((_WPW, _W), _f32),
            pltpu.VMEM((_W, F), jnp.bfloat16),
            pltpu.VMEM((_W, F), jnp.bfloat16),
            pltpu.VMEM((_W, F), jnp.bfloat16),
            pltpu.VMEM((_W, F), jnp.bfloat16),
            pltpu.VMEM((_W, F), _f32),
            pltpu.VMEM((_W, F), _f32),
            pltpu.VMEM((_W, F), _f32),
            pltpu.VMEM((_W, F), _f32),
            pltpu.VMEM((104, F), _f32),
            pltpu.VMEM_SHARED((_N, F), _f32),
        ] + [pltpu.SemaphoreType.DMA] * 8,
    )
    return k(hws, src.reshape(_NWIN, _W), dst.reshape(_NWIN, _W),
             ewn.reshape(_NWIN, _W))


def _edge1_kernel(hws_hbm, src_hbm, dst_hbm, ewn_hbm, m_hbm, out_hbm,
                  odinv_hbm, sidx_v, didx_v, ew_v, gb0, gb1, gb2, gb3,
                  r0, r1, r2, r3, dinv_v, m_v, acc_sh, deg_sh,
                  g0, g1, g2, g3, s0, s1, s2, s3):
    cid = lax.axis_index("c")
    sid = lax.axis_index("s")
    gbufs = (gb0, gb1, gb2, gb3)
    rows = (r0, r1, r2, r3)
    gsem = (g0, g1, g2, g3)
    ssem = (s0, s1, s2, s3)
    z16 = jnp.zeros((16,), _f32)

    pltpu.sync_copy(m_hbm.at[0, pl.ds(0, 16)], m_v)

    @pl.loop(0, _W)
    def _(r):
        for buf in rows:
            for c in range(4):
                buf[r, pl.ds(c * 16, 16)] = z16

    for q in range(4):
        pltpu.sync_copy(rows[q], acc_sh.at[pl.ds(sid * _RPS + q * _W, _W)])
    pltpu.sync_copy(r0.at[pl.ds(0, 112)],
                    acc_sh.at[pl.ds(sid * _RPS + 4 * _W, 112)])

    @pl.when(sid == 0)
    def _():
        pltpu.sync_copy(r0.at[pl.ds(0, 16)], acc_sh.at[pl.ds(16 * _RPS, 16)])

    @pl.when(sid < 5)
    def _():
        @pl.loop(0, 125)
        def _(r):
            dinv_v[pl.ds(r * 16, 16)] = z16
        pltpu.sync_copy(dinv_v.at[pl.ds(0, 2000)],
                        deg_sh.at[pl.ds(sid * 2000, 2000)])
    plsc.subcore_barrier()

    for ph in range(2):
        woff = (sid * 2 + ph) * _WPW
        pltpu.sync_copy(dst_hbm.at[pl.ds(woff, _WPW)], didx_v)
        pltpu.sync_copy(ewn_hbm.at[pl.ds(woff, _WPW)], ew_v)

        @pl.loop(0, _WPW, step=8)
        def _(w):
            for j in range(8):
                pltpu.async_copy(ew_v.at[w + j], deg_sh.at[didx_v.at[w + j]],
                                 g0, add=True)
            for j in range(8):
                pltpu.make_async_copy(ewn_hbm.at[0], ew_v.at[0], g0).wait()

    plsc.subcore_barrier()

    @pl.when(sid < 5)
    def _():
        sl = pl.ds(sid * 2000, 2000)
        pltpu.sync_copy(deg_sh.at[sl], dinv_v.at[pl.ds(0, 2000)])
        inv_m = 1.0 / m_v[...]
        magic = jnp.full((16,), 0x5F3759DF, _i32)

        @pl.loop(0, 125)
        def _(i):
            d = 1.0 + dinv_v[pl.ds(i * 16, 16)] * inv_m
            y = plsc.bitcast(
                magic - lax.shift_right_logical(plsc.bitcast(d, _i32), 1),
                _f32)
            for _it in range(3):
                y = y * (1.5 - 0.5 * d * y * y)
            dinv_v[pl.ds(i * 16, 16)] = y

        pltpu.sync_copy(dinv_v.at[pl.ds(0, 2000)], deg_sh.at[sl])
    plsc.subcore_barrier()

    pltpu.sync_copy(deg_sh, dinv_v)

    @pl.when((sid < 5) & (cid == 0))
    def _():
        sl = pl.ds(sid * 2000, 2000)
        pltpu.sync_copy(dinv_v.at[sl], odinv_hbm.at[sl])

    tbl = hws_hbm.at[cid]
    _edge_pipeline(tbl, src_hbm, dst_hbm, ewn_hbm, sid * 2 * _WPW, 4,
                   sidx_v, didx_v, ew_v, gbufs, rows, acc_sh, gsem, ssem,
                   dinv_v=dinv_v)
    _edge_pipeline(tbl, src_hbm, dst_hbm, ewn_hbm, (sid * 2 + 1) * _WPW, 4,
                   sidx_v, didx_v, ew_v, gbufs, rows, acc_sh, gsem, ssem,
                   dinv_v=dinv_v)
    plsc.subcore_barrier()

    sl = pl.ds(sid * _RPS, _RPS)
    pltpu.async_copy(acc_sh.at[sl], out_hbm.at[cid, sl], g0).wait()

    @pl.when(sid == 0)
    def _():
        tl = pl.ds(16 * _RPS, 16)
        pltpu.async_copy(acc_sh.at[tl], out_hbm.at[cid, tl], g0).wait()


def _edge1_call(hws_ab, src, dst, ewn, m):
    k = pl.kernel(
        _edge1_kernel,
        out_type=(jax.ShapeDtypeStruct((2, _N, 64), _f32),
                  jax.ShapeDtypeStruct((_N,), _f32)),
        mesh=_MESH,
        compiler_params=_CP,
        scratch_types=[
            pltpu.VMEM((_WPW, _W), _i32),
            pltpu.VMEM((_WPW, _W), _i32),
            pltpu.VMEM((_WPW, _W), _f32),
            pltpu.VMEM((_W, 64), jnp.bfloat16),
            pltpu.VMEM((_W, 64), jnp.bfloat16),
            pltpu.VMEM((_W, 64), jnp.bfloat16),
            pltpu.VMEM((_W, 64), jnp.bfloat16),
            pltpu.VMEM((_W, 64), _f32),
            pltpu.VMEM((_W, 64), _f32),
            pltpu.VMEM((_W, 64), _f32),
            pltpu.VMEM((_W, 64), _f32),
            pltpu.VMEM((_N,), _f32),
            pltpu.VMEM((16,), _f32),
            pltpu.VMEM_SHARED((_N, 64), _f32),
            pltpu.VMEM_SHARED((_N,), _f32),
        ] + [pltpu.SemaphoreType.DMA] * 8,
    )
    return k(hws_ab, src.reshape(_NWIN, _W), dst.reshape(_NWIN, _W),
             ewn.reshape(_NWIN, _W), m)


def kernel(x, edge_index, edge_attr, batch, W1, b1, W2, b2, W3, b3, Wlin, blin):
    pad = _EP - _E
    pad_idx = (jnp.arange(pad, dtype=_i32) * 64) % _N
    src = jnp.concatenate([edge_index[0], pad_idx])
    dst = jnp.concatenate([edge_index[1], pad_idx])
    ew = jnp.concatenate([edge_attr, jnp.zeros((pad,), _f32)])

    m, hw1, hw1b = pl.pallas_call(
        _pre_body,
        out_shape=(jax.ShapeDtypeStruct((8, 128), _f32),
                   jax.ShapeDtypeStruct((_N, 128), _f32),
                   jax.ShapeDtypeStruct((2, _N, 64), jnp.bfloat16)),
    )(edge_attr, x, W1)

    acc1, dinv = _edge1_call(hw1b, src, dst, ew, m)
    dcol = dinv.reshape(_N, 1)
    hws2, hws2b = pl.pallas_call(
        _layer1_body,
        out_shape=(jax.ShapeDtypeStruct((_N, 64), _f32),
                   jax.ShapeDtypeStruct((_N, 64), jnp.bfloat16)),
    )(acc1, hw1, dcol, m, b1, W2)
    a0, a1 = _edge_call(64, hws2b, src, dst, ew)
    hws3, hws3b = pl.pallas_call(
        _layer_body,
        out_shape=(jax.ShapeDtypeStruct((_N, 32), _f32),
                   jax.ShapeDtypeStruct((_N, 32), jnp.bfloat16)),
    )(a0, a1, hws2, dcol, m, b2, W3)
    a0, a1 = _edge_call(32, hws3b, src, dst, ew)

    out = _tc(_final_body, (_NG, _OUT))(
        a0, a1, hws3, dcol, m, b3, batch.reshape(1, _N), Wlin, blin)
    return out.reshape(_NG, 75, 16)
